# baseline (speedup 1.0000x reference)
import functools

import jax
import jax.numpy as jnp
from jax import lax
from jax.experimental import pallas as pl
from jax.experimental.pallas import tpu as pltpu
from jax.experimental.pallas import tpu_sc as plsc

B, L, D, DH, MM = 16, 512, 256, 1024, 4096
CHUNK = 128
HALF = MM // 2
NCH = HALF // CHUNK


def _shift_down(h):
    zr = jnp.zeros((1, h.shape[1]), h.dtype)
    return jnp.concatenate([zr, h[:-1, :]], axis=0)


def _shift_up(h):
    zr = jnp.zeros((1, h.shape[1]), h.dtype)
    return jnp.concatenate([h[1:, :], zr], axis=0)


def _conv3(h, w_ref, bias):
    h_hi = h.astype(jnp.bfloat16)
    acc = jnp.dot(_shift_down(h_hi), w_ref[0],
                  preferred_element_type=jnp.float32)
    acc = acc + jnp.dot(h_hi, w_ref[1], preferred_element_type=jnp.float32)
    acc = acc + jnp.dot(_shift_up(h_hi), w_ref[2],
                        preferred_element_type=jnp.float32)
    return acc + bias


def _ln(h, g, b):
    mu = jnp.mean(h, axis=-1, keepdims=True)
    xc = h - mu
    var = jnp.mean(xc * xc, axis=-1, keepdims=True)
    return xc * lax.rsqrt(var + 1e-5) * g + b


_BB = 8


def _predictor_body(x_ref, w1a, b1a, w2a, b2a, g1, be1,
                    w1b, b1b, w2b, b2b, g2, be2, wl, bl, dp_ref):
    for i in range(_BB):
        h = x_ref[i]
        h = jax.nn.relu(_conv3(h, w1a, b1a[0]))
        h = jax.nn.relu(_conv3(h, w2a, b2a[0]))
        h = _ln(h, g1[0], be1[0])
        h = jax.nn.relu(_conv3(h, w1b, b1b[0]))
        h = jax.nn.relu(_conv3(h, w2b, b2b[0]))
        h = _ln(h, g2[0], be2[0])
        dp = jnp.dot(h.astype(jnp.bfloat16), wl[...],
                     preferred_element_type=jnp.float32) + bl[0]
        dp_ref[0, i] = dp.reshape(1, L)[0]


def _predictor_call(x, w1a_t, b1a, w2a_t, b2a, g1, be1,
                    w1b_t, b1b, w2b_t, b2b, g2, be2, wl, bl):
    full = lambda shape: pl.BlockSpec(shape, lambda b: (0,) * len(shape))
    return pl.pallas_call(
        _predictor_body,
        grid=(B // _BB,),
        in_specs=[
            pl.BlockSpec((_BB, L, D), lambda b: (b, 0, 0)),
            full((3, D, DH)), full((1, DH)),
            full((3, DH, D)), full((1, D)),
            full((1, D)), full((1, D)),
            full((3, D, DH)), full((1, DH)),
            full((3, DH, D)), full((1, D)),
            full((1, D)), full((1, D)),
            full((D, 1)), full((1, 1)),
        ],
        out_specs=pl.BlockSpec((1, _BB, L), lambda b: (b, 0, 0)),
        out_shape=jax.ShapeDtypeStruct((B // _BB, _BB, L), jnp.float32),
    )(x, w1a_t, b1a, w2a_t, b2a, g1, be1,
      w1b_t, b1b, w2b_t, b2b, g2, be2, wl, bl)


def _regulate(x_flat, tgt):
    mesh = plsc.VectorSubcoreMesh(core_axis_name="c", subcore_axis_name="s")

    @functools.partial(
        pl.kernel,
        mesh=mesh,
        out_type=jax.ShapeDtypeStruct((B * MM, D), jnp.float32),
        compiler_params=pltpu.CompilerParams(needs_layout_passes=False),
        scratch_types=[
            pltpu.VMEM((L,), jnp.int32),
            pltpu.VMEM((MM,), jnp.int32),
            pltpu.VMEM((HALF,), jnp.int32),
            [pltpu.VMEM((CHUNK, D), jnp.float32) for _ in range(2)],
            pltpu.VMEM((CHUNK, D), jnp.float32),
            [pltpu.SemaphoreType.DMA for _ in range(2)],
            [pltpu.SemaphoreType.DMA for _ in range(2)],
        ],
    )
    def k(x_hbm, tgt_hbm, out_hbm, dur_v, a_v, idx_v, bufs, zbuf,
          gsems, wsems):
        b_ = lax.axis_index("s")
        h_ = lax.axis_index("c")

        pltpu.sync_copy(tgt_hbm.at[b_], dur_v)

        def z_body(j, carry):
            a_v[pl.ds(j * 16, 16)] = jnp.zeros((16,), jnp.int32)
            return carry
        lax.fori_loop(0, MM // 16, z_body, 0)

        def s_body(i, carry):
            dch = dur_v[pl.ds(i * 16, 16)]
            ech = plsc.cumsum(dch) + carry
            sch = ech - dch
            vals = lax.iota(jnp.int32, 16) + (i * 16 + 1)
            plsc.store_scatter(a_v, [sch], vals, mask=dch > 0)
            return carry + jnp.sum(dch)
        total = lax.fori_loop(0, L // 16, s_body, jnp.int32(0))

        lane0 = lax.iota(jnp.int32, 16) == 0
        plsc.store_scatter(a_v, [jnp.zeros((16,), jnp.int32) + total],
                           jnp.full((16,), L + 1, jnp.int32), mask=lane0)

        base = b_ * L
        def c_body(j, carry):
            a = a_v[pl.ds(j * 16, 16)]
            cm = jnp.maximum(plsc.cummax(a), carry)

            @pl.when(((j >> 3) & 1) == h_)
            def _():
                idx = jnp.where(cm == L + 1, base, base + cm - 1)
                idx_v[pl.ds(((j >> 4) << 7) + ((j & 7) << 4), 16)] = idx
            return jnp.maximum(carry, jnp.max(a))
        lax.fori_loop(0, MM // 16, c_body, jnp.int32(0))

        def f0(c):
            return (2 * c + h_) * CHUNK

        def g_start(g):
            @pl.when(f0(g) < total)
            def _():
                pltpu.async_copy(
                    x_hbm.at[idx_v.at[pl.ds(g * CHUNK, CHUNK)]],
                    bufs[g % 2], gsems[g % 2])

        def g_wait(g):
            @pl.when(f0(g) < total)
            def _():
                pltpu.make_async_copy(
                    x_hbm.at[idx_v.at[pl.ds(g * CHUNK, CHUNK)]],
                    bufs[g % 2], gsems[g % 2]).wait()
                lo = jnp.clip(total - f0(g), 0, CHUNK) * 16

                def zt_body(j, carry):
                    bufs[g % 2][j >> 4, pl.ds((j & 15) * 16, 16)] = (
                        jnp.zeros((16,), jnp.float32))
                    return carry
                lax.fori_loop(lo, CHUNK * 16, zt_body, 0)

        def w_start(c):
            dst = out_hbm.at[pl.ds(b_ * MM + f0(c), CHUNK)]

            @pl.when(f0(c) < total)
            def _():
                pltpu.make_async_copy(bufs[c % 2], dst, wsems[c % 2]).start()

            @pl.when(f0(c) >= total)
            def _():
                pltpu.make_async_copy(zbuf, dst, wsems[c % 2]).start()

        def w_wait(c):
            pltpu.make_async_copy(
                bufs[c % 2], out_hbm.at[pl.ds(b_ * MM + f0(c), CHUNK)],
                wsems[c % 2]).wait()

        g_start(0)

        def zb_body(j, carry):
            zbuf[j >> 4, pl.ds((j & 15) * 16, 16)] = jnp.zeros((16,),
                                                               jnp.float32)
            return carry
        lax.fori_loop(0, CHUNK * D // 16, zb_body, 0)

        for c in range(NCH):
            g_wait(c)
            w_start(c)
            if c + 1 < NCH:
                if c >= 1:
                    w_wait(c - 1)
                g_start(c + 1)
        w_wait(NCH - 2)
        w_wait(NCH - 1)

    return k(x_flat, tgt)


def kernel(x, target, mel_max_len, W1a, b1a, W2a, b2a, ln1_g, ln1_b,
           W1b, b1b, W2b, b2b, ln2_g, ln2_b, Wlin, blin):
    x = x.astype(jnp.float32)
    tgt = target.astype(jnp.int32)

    out_flat = _regulate(x.reshape(B * L, D), tgt)

    row = lambda v: v.reshape(1, -1)
    tbf = lambda w: jnp.transpose(w.astype(jnp.bfloat16), (2, 1, 0))
    dp = _predictor_call(
        x,
        tbf(W1a), row(b1a),
        tbf(W2a), row(b2a),
        row(ln1_g), row(ln1_b),
        tbf(W1b), row(b1b),
        tbf(W2b), row(b2b),
        row(ln2_g), row(ln2_b),
        Wlin.astype(jnp.bfloat16), row(blin),
    )
    return out_flat.reshape(B, MM, D), dp.reshape(B, L)


# speedup vs baseline: 1.2699x; 1.2699x over previous
import functools

import jax
import jax.numpy as jnp
from jax import lax
from jax.experimental import pallas as pl
from jax.experimental.pallas import tpu as pltpu
from jax.experimental.pallas import tpu_sc as plsc

B, L, D, DH, MM = 16, 512, 256, 1024, 4096
CHUNK = 128
HALF = MM // 2
NCH = HALF // CHUNK


def _shift_down(h):
    zr = jnp.zeros((1, h.shape[1]), h.dtype)
    return jnp.concatenate([zr, h[:-1, :]], axis=0)


def _shift_up(h):
    zr = jnp.zeros((1, h.shape[1]), h.dtype)
    return jnp.concatenate([h[1:, :], zr], axis=0)


def _conv3(h, w_ref, bias):
    h_hi = h.astype(jnp.bfloat16)
    acc = jnp.dot(_shift_down(h_hi), w_ref[0],
                  preferred_element_type=jnp.float32)
    acc = acc + jnp.dot(h_hi, w_ref[1], preferred_element_type=jnp.float32)
    acc = acc + jnp.dot(_shift_up(h_hi), w_ref[2],
                        preferred_element_type=jnp.float32)
    return acc + bias


def _ln(h, g, b):
    mu = jnp.mean(h, axis=-1, keepdims=True)
    xc = h - mu
    var = jnp.mean(xc * xc, axis=-1, keepdims=True)
    return xc * lax.rsqrt(var + 1e-5) * g + b


_BB = 4


def _predictor_body(x_ref, w1a, b1a, w2a, b2a, g1, be1,
                    w1b, b1b, w2b, b2b, g2, be2, wl, bl, dp_ref):
    for i in range(_BB):
        h = x_ref[i]
        h = jax.nn.relu(_conv3(h, w1a, b1a[0]))
        h = jax.nn.relu(_conv3(h, w2a, b2a[0]))
        h = _ln(h, g1[0], be1[0])
        h = jax.nn.relu(_conv3(h, w1b, b1b[0]))
        h = jax.nn.relu(_conv3(h, w2b, b2b[0]))
        h = _ln(h, g2[0], be2[0])
        dp = jnp.dot(h.astype(jnp.bfloat16), wl[...],
                     preferred_element_type=jnp.float32) + bl[0]
        dp_ref[0, i] = dp.reshape(1, L)[0]


def _predictor_call(x, w1a_t, b1a, w2a_t, b2a, g1, be1,
                    w1b_t, b1b, w2b_t, b2b, g2, be2, wl, bl):
    full = lambda shape: pl.BlockSpec(shape, lambda b: (0,) * len(shape))
    return pl.pallas_call(
        _predictor_body,
        grid=(B // _BB,),
        in_specs=[
            pl.BlockSpec((_BB, L, D), lambda b: (b, 0, 0)),
            full((3, D, DH)), full((1, DH)),
            full((3, DH, D)), full((1, D)),
            full((1, D)), full((1, D)),
            full((3, D, DH)), full((1, DH)),
            full((3, DH, D)), full((1, D)),
            full((1, D)), full((1, D)),
            full((D, 1)), full((1, 1)),
        ],
        out_specs=pl.BlockSpec((1, _BB, L), lambda b: (b, 0, 0)),
        out_shape=jax.ShapeDtypeStruct((B // _BB, _BB, L), jnp.float32),
    )(x, w1a_t, b1a, w2a_t, b2a, g1, be1,
      w1b_t, b1b, w2b_t, b2b, g2, be2, wl, bl)


def _regulate(x_flat, tgt):
    mesh = plsc.VectorSubcoreMesh(core_axis_name="c", subcore_axis_name="s")

    @functools.partial(
        pl.kernel,
        mesh=mesh,
        out_type=jax.ShapeDtypeStruct((B * MM, D), jnp.float32),
        compiler_params=pltpu.CompilerParams(needs_layout_passes=False),
        scratch_types=[
            pltpu.VMEM((L,), jnp.int32),
            pltpu.VMEM((MM,), jnp.int32),
            pltpu.VMEM((HALF,), jnp.int32),
            [pltpu.VMEM((CHUNK, D), jnp.float32) for _ in range(2)],
            pltpu.VMEM((CHUNK, D), jnp.float32),
            [pltpu.SemaphoreType.DMA for _ in range(2)],
            [pltpu.SemaphoreType.DMA for _ in range(2)],
        ],
    )
    def k(x_hbm, tgt_hbm, out_hbm, dur_v, a_v, idx_v, bufs, zbuf,
          gsems, wsems):
        b_ = lax.axis_index("s")
        h_ = lax.axis_index("c")

        pltpu.sync_copy(tgt_hbm.at[b_], dur_v)

        def z_body(j, carry):
            a_v[pl.ds(j * 16, 16)] = jnp.zeros((16,), jnp.int32)
            return carry
        lax.fori_loop(0, MM // 16, z_body, 0)

        def s_body(i, carry):
            dch = dur_v[pl.ds(i * 16, 16)]
            ech = plsc.cumsum(dch) + carry
            sch = ech - dch
            vals = lax.iota(jnp.int32, 16) + (i * 16 + 1)
            plsc.store_scatter(a_v, [sch], vals, mask=dch > 0)
            return carry + jnp.sum(dch)
        total = lax.fori_loop(0, L // 16, s_body, jnp.int32(0))

        lane0 = lax.iota(jnp.int32, 16) == 0
        plsc.store_scatter(a_v, [jnp.zeros((16,), jnp.int32) + total],
                           jnp.full((16,), L + 1, jnp.int32), mask=lane0)

        base = b_ * L
        def c_body(j, carry):
            a = a_v[pl.ds(j * 16, 16)]
            cm = jnp.maximum(plsc.cummax(a), carry)

            @pl.when(((j >> 3) & 1) == h_)
            def _():
                idx = jnp.where(cm == L + 1, base, base + cm - 1)
                idx_v[pl.ds(((j >> 4) << 7) + ((j & 7) << 4), 16)] = idx
            return jnp.maximum(carry, jnp.max(a))
        lax.fori_loop(0, MM // 16, c_body, jnp.int32(0))

        def f0(c):
            return (2 * c + h_) * CHUNK

        def g_start(g):
            @pl.when(f0(g) < total)
            def _():
                pltpu.async_copy(
                    x_hbm.at[idx_v.at[pl.ds(g * CHUNK, CHUNK)]],
                    bufs[g % 2], gsems[g % 2])

        def g_wait(g):
            @pl.when(f0(g) < total)
            def _():
                pltpu.make_async_copy(
                    x_hbm.at[idx_v.at[pl.ds(g * CHUNK, CHUNK)]],
                    bufs[g % 2], gsems[g % 2]).wait()
                lo = jnp.clip(total - f0(g), 0, CHUNK) * 16

                def zt_body(j, carry):
                    bufs[g % 2][j >> 4, pl.ds((j & 15) * 16, 16)] = (
                        jnp.zeros((16,), jnp.float32))
                    return carry
                lax.fori_loop(lo, CHUNK * 16, zt_body, 0)

        def w_start(c):
            dst = out_hbm.at[pl.ds(b_ * MM + f0(c), CHUNK)]

            @pl.when(f0(c) < total)
            def _():
                pltpu.make_async_copy(bufs[c % 2], dst, wsems[c % 2]).start()

            @pl.when(f0(c) >= total)
            def _():
                pltpu.make_async_copy(zbuf, dst, wsems[c % 2]).start()

        def w_wait(c):
            pltpu.make_async_copy(
                bufs[c % 2], out_hbm.at[pl.ds(b_ * MM + f0(c), CHUNK)],
                wsems[c % 2]).wait()

        g_start(0)

        def zb_body(j, carry):
            zbuf[j >> 4, pl.ds((j & 15) * 16, 16)] = jnp.zeros((16,),
                                                               jnp.float32)
            return carry
        lax.fori_loop(0, CHUNK * D // 16, zb_body, 0)

        for c in range(NCH):
            g_wait(c)
            w_start(c)
            if c + 1 < NCH:
                if c >= 1:
                    w_wait(c - 1)
                g_start(c + 1)
        w_wait(NCH - 2)
        w_wait(NCH - 1)

    return k(x_flat, tgt)


def kernel(x, target, mel_max_len, W1a, b1a, W2a, b2a, ln1_g, ln1_b,
           W1b, b1b, W2b, b2b, ln2_g, ln2_b, Wlin, blin):
    x = x.astype(jnp.float32)
    tgt = target.astype(jnp.int32)

    out_flat = _regulate(x.reshape(B * L, D), tgt)

    row = lambda v: v.reshape(1, -1)
    tbf = lambda w: jnp.transpose(w.astype(jnp.bfloat16), (2, 1, 0))
    dp = _predictor_call(
        x,
        tbf(W1a), row(b1a),
        tbf(W2a), row(b2a),
        row(ln1_g), row(ln1_b),
        tbf(W1b), row(b1b),
        tbf(W2b), row(b2b),
        row(ln2_g), row(ln2_b),
        Wlin.astype(jnp.bfloat16), row(blin),
    )
    return out_flat.reshape(B, MM, D), dp.reshape(B, L)


# speedup vs baseline: 1.5147x; 1.1927x over previous
import functools

import jax
import jax.numpy as jnp
from jax import lax
from jax.experimental import pallas as pl
from jax.experimental.pallas import tpu as pltpu
from jax.experimental.pallas import tpu_sc as plsc

B, L, D, DH, MM = 16, 512, 256, 1024, 4096
CHUNK = 128
HALF = MM // 2
NCH = HALF // CHUNK


def _shift_down(h):
    zr = jnp.zeros((1, h.shape[1]), h.dtype)
    return jnp.concatenate([zr, h[:-1, :]], axis=0)


def _shift_up(h):
    zr = jnp.zeros((1, h.shape[1]), h.dtype)
    return jnp.concatenate([h[1:, :], zr], axis=0)


def _conv3(h, w_ref, bias):
    h_hi = h.astype(jnp.bfloat16)
    yn = jnp.dot(h_hi, w_ref[...], preferred_element_type=jnp.float32)
    n = yn.shape[1] // 3
    y = yn[:, n:2 * n] + _shift_down(yn[:, :n]) + _shift_up(yn[:, 2 * n:])
    return y + bias


def _ln(h, g, b):
    mu = jnp.mean(h, axis=-1, keepdims=True)
    xc = h - mu
    var = jnp.mean(xc * xc, axis=-1, keepdims=True)
    return xc * lax.rsqrt(var + 1e-5) * g + b


_BB = 4


def _predictor_body(x_ref, w1a, b1a, w2a, b2a, g1, be1,
                    w1b, b1b, w2b, b2b, g2, be2, wl, bl, dp_ref):
    for i in range(_BB):
        h = x_ref[i]
        h = jax.nn.relu(_conv3(h, w1a, b1a[0]))
        h = jax.nn.relu(_conv3(h, w2a, b2a[0]))
        h = _ln(h, g1[0], be1[0])
        h = jax.nn.relu(_conv3(h, w1b, b1b[0]))
        h = jax.nn.relu(_conv3(h, w2b, b2b[0]))
        h = _ln(h, g2[0], be2[0])
        dp = jnp.dot(h.astype(jnp.bfloat16), wl[...],
                     preferred_element_type=jnp.float32) + bl[0]
        dp_ref[0, i] = dp.reshape(1, L)[0]


def _predictor_call(x, w1a_t, b1a, w2a_t, b2a, g1, be1,
                    w1b_t, b1b, w2b_t, b2b, g2, be2, wl, bl):
    full = lambda shape: pl.BlockSpec(shape, lambda b: (0,) * len(shape))
    return pl.pallas_call(
        _predictor_body,
        grid=(B // _BB,),
        in_specs=[
            pl.BlockSpec((_BB, L, D), lambda b: (b, 0, 0)),
            full((D, 3 * DH)), full((1, DH)),
            full((DH, 3 * D)), full((1, D)),
            full((1, D)), full((1, D)),
            full((D, 3 * DH)), full((1, DH)),
            full((DH, 3 * D)), full((1, D)),
            full((1, D)), full((1, D)),
            full((D, 1)), full((1, 1)),
        ],
        out_specs=pl.BlockSpec((1, _BB, L), lambda b: (b, 0, 0)),
        out_shape=jax.ShapeDtypeStruct((B // _BB, _BB, L), jnp.float32),
    )(x, w1a_t, b1a, w2a_t, b2a, g1, be1,
      w1b_t, b1b, w2b_t, b2b, g2, be2, wl, bl)


def _regulate(x_flat, tgt):
    mesh = plsc.VectorSubcoreMesh(core_axis_name="c", subcore_axis_name="s")

    @functools.partial(
        pl.kernel,
        mesh=mesh,
        out_type=jax.ShapeDtypeStruct((B * MM, D), jnp.float32),
        compiler_params=pltpu.CompilerParams(needs_layout_passes=False),
        scratch_types=[
            pltpu.VMEM((L,), jnp.int32),
            pltpu.VMEM((MM,), jnp.int32),
            pltpu.VMEM((HALF,), jnp.int32),
            [pltpu.VMEM((CHUNK, D), jnp.float32) for _ in range(2)],
            pltpu.VMEM((CHUNK, D), jnp.float32),
            [pltpu.SemaphoreType.DMA for _ in range(2)],
            [pltpu.SemaphoreType.DMA for _ in range(2)],
        ],
    )
    def k(x_hbm, tgt_hbm, out_hbm, dur_v, a_v, idx_v, bufs, zbuf,
          gsems, wsems):
        b_ = lax.axis_index("s")
        h_ = lax.axis_index("c")

        pltpu.sync_copy(tgt_hbm.at[b_], dur_v)

        def z_body(j, carry):
            a_v[pl.ds(j * 16, 16)] = jnp.zeros((16,), jnp.int32)
            return carry
        lax.fori_loop(0, MM // 16, z_body, 0)

        def s_body(i, carry):
            dch = dur_v[pl.ds(i * 16, 16)]
            ech = plsc.cumsum(dch) + carry
            sch = ech - dch
            vals = lax.iota(jnp.int32, 16) + (i * 16 + 1)
            plsc.store_scatter(a_v, [sch], vals, mask=dch > 0)
            return carry + jnp.sum(dch)
        total = lax.fori_loop(0, L // 16, s_body, jnp.int32(0))

        lane0 = lax.iota(jnp.int32, 16) == 0
        plsc.store_scatter(a_v, [jnp.zeros((16,), jnp.int32) + total],
                           jnp.full((16,), L + 1, jnp.int32), mask=lane0)

        base = b_ * L
        def c_body(j, carry):
            a = a_v[pl.ds(j * 16, 16)]
            cm = jnp.maximum(plsc.cummax(a), carry)

            @pl.when(((j >> 3) & 1) == h_)
            def _():
                idx = jnp.where(cm == L + 1, base, base + cm - 1)
                idx_v[pl.ds(((j >> 4) << 7) + ((j & 7) << 4), 16)] = idx
            return jnp.maximum(carry, jnp.max(a))
        lax.fori_loop(0, MM // 16, c_body, jnp.int32(0))

        def f0(c):
            return (2 * c + h_) * CHUNK

        def g_start(g):
            @pl.when(f0(g) < total)
            def _():
                pltpu.async_copy(
                    x_hbm.at[idx_v.at[pl.ds(g * CHUNK, CHUNK)]],
                    bufs[g % 2], gsems[g % 2])

        def g_wait(g):
            @pl.when(f0(g) < total)
            def _():
                pltpu.make_async_copy(
                    x_hbm.at[idx_v.at[pl.ds(g * CHUNK, CHUNK)]],
                    bufs[g % 2], gsems[g % 2]).wait()
                lo = jnp.clip(total - f0(g), 0, CHUNK) * 16

                def zt_body(j, carry):
                    bufs[g % 2][j >> 4, pl.ds((j & 15) * 16, 16)] = (
                        jnp.zeros((16,), jnp.float32))
                    return carry
                lax.fori_loop(lo, CHUNK * 16, zt_body, 0)

        def w_start(c):
            dst = out_hbm.at[pl.ds(b_ * MM + f0(c), CHUNK)]

            @pl.when(f0(c) < total)
            def _():
                pltpu.make_async_copy(bufs[c % 2], dst, wsems[c % 2]).start()

            @pl.when(f0(c) >= total)
            def _():
                pltpu.make_async_copy(zbuf, dst, wsems[c % 2]).start()

        def w_wait(c):
            pltpu.make_async_copy(
                bufs[c % 2], out_hbm.at[pl.ds(b_ * MM + f0(c), CHUNK)],
                wsems[c % 2]).wait()

        g_start(0)

        def zb_body(j, carry):
            zbuf[j >> 4, pl.ds((j & 15) * 16, 16)] = jnp.zeros((16,),
                                                               jnp.float32)
            return carry
        lax.fori_loop(0, CHUNK * D // 16, zb_body, 0)

        for c in range(NCH):
            g_wait(c)
            w_start(c)
            if c + 1 < NCH:
                if c >= 1:
                    w_wait(c - 1)
                g_start(c + 1)
        w_wait(NCH - 2)
        w_wait(NCH - 1)

    return k(x_flat, tgt)


def kernel(x, target, mel_max_len, W1a, b1a, W2a, b2a, ln1_g, ln1_b,
           W1b, b1b, W2b, b2b, ln2_g, ln2_b, Wlin, blin):
    x = x.astype(jnp.float32)
    tgt = target.astype(jnp.int32)

    out_flat = _regulate(x.reshape(B * L, D), tgt)

    row = lambda v: v.reshape(1, -1)
    tbf = lambda w: jnp.transpose(w.astype(jnp.bfloat16), (1, 2, 0)).reshape(
        w.shape[1], 3 * w.shape[0])
    dp = _predictor_call(
        x,
        tbf(W1a), row(b1a),
        tbf(W2a), row(b2a),
        row(ln1_g), row(ln1_b),
        tbf(W1b), row(b1b),
        tbf(W2b), row(b2b),
        row(ln2_g), row(ln2_b),
        Wlin.astype(jnp.bfloat16), row(blin),
    )
    return out_flat.reshape(B, MM, D), dp.reshape(B, L)


# speedup vs baseline: 1.5193x; 1.0031x over previous
import functools

import jax
import jax.numpy as jnp
from jax import lax
from jax.experimental import pallas as pl
from jax.experimental.pallas import tpu as pltpu
from jax.experimental.pallas import tpu_sc as plsc

B, L, D, DH, MM = 16, 512, 256, 1024, 4096
CHUNK = 128
HALF = MM // 2
NCH = HALF // CHUNK


def _shift_down(h):
    zr = jnp.zeros((1, h.shape[1]), h.dtype)
    return jnp.concatenate([zr, h[:-1, :]], axis=0)


def _shift_up(h):
    zr = jnp.zeros((1, h.shape[1]), h.dtype)
    return jnp.concatenate([h[1:, :], zr], axis=0)


def _conv3(h, w_ref, bias):
    h_hi = h.astype(jnp.bfloat16)
    yn = jnp.dot(h_hi, w_ref[...], preferred_element_type=jnp.float32)
    n = yn.shape[1] // 3
    y = yn[:, n:2 * n] + _shift_down(yn[:, :n]) + _shift_up(yn[:, 2 * n:])
    return y + bias


def _ln(h, g, b):
    mu = jnp.mean(h, axis=-1, keepdims=True)
    xc = h - mu
    var = jnp.mean(xc * xc, axis=-1, keepdims=True)
    return xc * lax.rsqrt(var + 1e-5) * g + b


_BB = 4


def _predictor_body(x_ref, w1a, b1a, w2a, b2a, g1, be1,
                    w1b, b1b, w2b, b2b, g2, be2, wl, bl, dp_ref):
    for i in range(_BB):
        h = x_ref[i]
        h = jax.nn.relu(_conv3(h, w1a, b1a[0]))
        h = jax.nn.relu(_conv3(h, w2a, b2a[0]))
        h = _ln(h, g1[0], be1[0])
        h = jax.nn.relu(_conv3(h, w1b, b1b[0]))
        h = jax.nn.relu(_conv3(h, w2b, b2b[0]))
        h = _ln(h, g2[0], be2[0])
        dp = jnp.dot(h.astype(jnp.bfloat16), wl[...],
                     preferred_element_type=jnp.float32) + bl[0]
        dp_ref[0, i] = dp.reshape(1, L)[0]


def _predictor_call(x, w1a_t, b1a, w2a_t, b2a, g1, be1,
                    w1b_t, b1b, w2b_t, b2b, g2, be2, wl, bl):
    full = lambda shape: pl.BlockSpec(shape, lambda b: (0,) * len(shape))
    return pl.pallas_call(
        _predictor_body,
        grid=(B // _BB,),
        in_specs=[
            pl.BlockSpec((_BB, L, D), lambda b: (b, 0, 0)),
            full((D, 3 * DH)), full((1, DH)),
            full((DH, 3 * D)), full((1, D)),
            full((1, D)), full((1, D)),
            full((D, 3 * DH)), full((1, DH)),
            full((DH, 3 * D)), full((1, D)),
            full((1, D)), full((1, D)),
            full((D, 1)), full((1, 1)),
        ],
        out_specs=pl.BlockSpec((1, _BB, L), lambda b: (b, 0, 0)),
        out_shape=jax.ShapeDtypeStruct((B // _BB, _BB, L), jnp.float32),
    )(x, w1a_t, b1a, w2a_t, b2a, g1, be1,
      w1b_t, b1b, w2b_t, b2b, g2, be2, wl, bl)


def _regulate(x_flat, tgt):
    mesh = plsc.VectorSubcoreMesh(core_axis_name="c", subcore_axis_name="s")

    @functools.partial(
        pl.kernel,
        mesh=mesh,
        out_type=jax.ShapeDtypeStruct((B, MM, D), jnp.float32),
        compiler_params=pltpu.CompilerParams(needs_layout_passes=False),
        scratch_types=[
            pltpu.VMEM((L,), jnp.int32),
            pltpu.VMEM((MM,), jnp.int32),
            pltpu.VMEM((HALF,), jnp.int32),
            [pltpu.VMEM((CHUNK, D), jnp.float32) for _ in range(2)],
            pltpu.VMEM((CHUNK, D), jnp.float32),
            [pltpu.SemaphoreType.DMA for _ in range(2)],
            [pltpu.SemaphoreType.DMA for _ in range(2)],
        ],
    )
    def k(x_hbm, tgt_hbm, out_hbm, dur_v, a_v, idx_v, bufs, zbuf,
          gsems, wsems):
        b_ = lax.axis_index("s")
        h_ = lax.axis_index("c")

        pltpu.sync_copy(tgt_hbm.at[b_], dur_v)

        def z_body(j, carry):
            a_v[pl.ds(j * 16, 16)] = jnp.zeros((16,), jnp.int32)
            return carry
        lax.fori_loop(0, MM // 16, z_body, 0)

        def s_body(i, carry):
            dch = dur_v[pl.ds(i * 16, 16)]
            ech = plsc.cumsum(dch) + carry
            sch = ech - dch
            vals = lax.iota(jnp.int32, 16) + (i * 16 + 1)
            plsc.store_scatter(a_v, [sch], vals, mask=dch > 0)
            return carry + jnp.sum(dch)
        total = lax.fori_loop(0, L // 16, s_body, jnp.int32(0))

        lane0 = lax.iota(jnp.int32, 16) == 0
        plsc.store_scatter(a_v, [jnp.zeros((16,), jnp.int32) + total],
                           jnp.full((16,), L + 1, jnp.int32), mask=lane0)

        base = b_ * L
        def c_body(j, carry):
            a = a_v[pl.ds(j * 16, 16)]
            cm = jnp.maximum(plsc.cummax(a), carry)

            @pl.when(((j >> 3) & 1) == h_)
            def _():
                idx = jnp.where(cm == L + 1, base, base + cm - 1)
                idx_v[pl.ds(((j >> 4) << 7) + ((j & 7) << 4), 16)] = idx
            return jnp.maximum(carry, jnp.max(a))
        lax.fori_loop(0, MM // 16, c_body, jnp.int32(0))

        def f0(c):
            return (2 * c + h_) * CHUNK

        def g_start(g):
            @pl.when(f0(g) < total)
            def _():
                pltpu.async_copy(
                    x_hbm.at[idx_v.at[pl.ds(g * CHUNK, CHUNK)]],
                    bufs[g % 2], gsems[g % 2])

        def g_wait(g):
            @pl.when(f0(g) < total)
            def _():
                pltpu.make_async_copy(
                    x_hbm.at[idx_v.at[pl.ds(g * CHUNK, CHUNK)]],
                    bufs[g % 2], gsems[g % 2]).wait()
                lo = jnp.clip(total - f0(g), 0, CHUNK) * 16

                def zt_body(j, carry):
                    bufs[g % 2][j >> 4, pl.ds((j & 15) * 16, 16)] = (
                        jnp.zeros((16,), jnp.float32))
                    return carry
                lax.fori_loop(lo, CHUNK * 16, zt_body, 0)

        def w_start(c):
            dst = out_hbm.at[b_, pl.ds(f0(c), CHUNK)]

            @pl.when(f0(c) < total)
            def _():
                pltpu.make_async_copy(bufs[c % 2], dst, wsems[c % 2]).start()

            @pl.when(f0(c) >= total)
            def _():
                pltpu.make_async_copy(zbuf, dst, wsems[c % 2]).start()

        def w_wait(c):
            pltpu.make_async_copy(
                bufs[c % 2], out_hbm.at[b_, pl.ds(f0(c), CHUNK)],
                wsems[c % 2]).wait()

        g_start(0)

        def zb_body(j, carry):
            zbuf[j >> 4, pl.ds((j & 15) * 16, 16)] = jnp.zeros((16,),
                                                               jnp.float32)
            return carry
        lax.fori_loop(0, CHUNK * D // 16, zb_body, 0)

        for c in range(NCH):
            g_wait(c)
            w_start(c)
            if c + 1 < NCH:
                if c >= 1:
                    w_wait(c - 1)
                g_start(c + 1)
        w_wait(NCH - 2)
        w_wait(NCH - 1)

    return k(x_flat, tgt)


def kernel(x, target, mel_max_len, W1a, b1a, W2a, b2a, ln1_g, ln1_b,
           W1b, b1b, W2b, b2b, ln2_g, ln2_b, Wlin, blin):
    x = x.astype(jnp.float32)
    tgt = target.astype(jnp.int32)

    out = _regulate(x.reshape(B * L, D), tgt)

    row = lambda v: v.reshape(1, -1)
    tbf = lambda w: jnp.transpose(w.astype(jnp.bfloat16), (1, 2, 0)).reshape(
        w.shape[1], 3 * w.shape[0])
    dp = _predictor_call(
        x,
        tbf(W1a), row(b1a),
        tbf(W2a), row(b2a),
        row(ln1_g), row(ln1_b),
        tbf(W1b), row(b1b),
        tbf(W2b), row(b2b),
        row(ln2_g), row(ln2_b),
        Wlin.astype(jnp.bfloat16), row(blin),
    )
    return out, dp.reshape(B, L)
